# baseline (speedup 1.0000x reference)
import functools

import jax
import jax.numpy as jnp
from jax import lax
from jax.experimental import pallas as pl
from jax.experimental.pallas import tpu as pltpu
from jax.experimental.pallas import tpu_sc as plsc

N_CLS = 1024
L_SUF = 61
L_PAD = 64
N_CTX = 16
CTX_LEN = 77
D = 512
NC, NS = 2, 16
NW = NC * NS
CPW = N_CLS // NW
NBUF = 3


def _body(prompt_hbm, promptT_hbm, ctx_hbm, table_hbm, out_hbm, eos_hbm,
          pbuf, pbufT, rows, eos_v, gsems, wsems, csem):
    wid = lax.axis_index("s") * NC + lax.axis_index("c")
    base = wid * CPW
    pltpu.sync_copy(prompt_hbm.at[pl.ds(base, CPW)], pbuf)

    def issue_gather(t, b):
        pltpu.async_copy(table_hbm.at[pbuf.at[t]], rows[b], gsems[b])

    def wait_gather(t, b):
        pltpu.make_async_copy(table_hbm.at[pbuf.at[t]], rows[b],
                              gsems[b]).wait()

    def issue_writes(t, b):
        o = (base + t) * CTX_LEN
        pltpu.async_copy(rows[b].at[pl.ds(0, 1)],
                         out_hbm.at[pl.ds(o, 1)], wsems[b])
        pltpu.async_copy(rows[b].at[pl.ds(1, L_SUF - 1)],
                         out_hbm.at[pl.ds(o + 1 + N_CTX, L_SUF - 1)], wsems[b])

    def drain_writes(b):
        pltpu.make_async_copy(rows[b].at[pl.ds(0, L_SUF)],
                              out_hbm.at[pl.ds(0, L_SUF)], wsems[b]).wait()

    def issue_ctx(t):
        c = base + t
        pltpu.async_copy(ctx_hbm.at[pl.ds(c * N_CTX, N_CTX)],
                         out_hbm.at[pl.ds(c * CTX_LEN + 1, N_CTX)], csem)

    issue_gather(0, 0)

    pltpu.sync_copy(promptT_hbm.at[:, pl.ds(base, CPW)], pbufT)
    for h in range(CPW // 16):
        def amax(j, carry):
            best, besti = carry
            cur = pbufT[j, pl.ds(h * 16, 16)]
            m = cur > best
            return (jnp.where(m, cur, best),
                    jnp.where(m, jnp.full((16,), 1, jnp.int32) * j, besti))

        init = (jnp.full((16,), -1, jnp.int32), jnp.zeros((16,), jnp.int32))
        _, besti = lax.fori_loop(0, L_SUF, amax, init)
        eos_v[pl.ds(h * 16, 16)] = besti + N_CTX
    pltpu.sync_copy(eos_v, eos_hbm.at[pl.ds(base, CPW)])

    for t in range(CPW):
        b = t % NBUF
        wait_gather(t, b)
        issue_writes(t, b)
        issue_ctx(t)
        u = t + 1
        if u < CPW:
            bu = u % NBUF
            if u >= NBUF:
                drain_writes(bu)
            issue_gather(u, bu)

    for t in range(CPW - NBUF + 1, CPW):
        drain_writes(t % NBUF)
    pltpu.make_async_copy(ctx_hbm.at[pl.ds(0, CPW * N_CTX)],
                          out_hbm.at[pl.ds(0, CPW * N_CTX)], csem).wait()


@functools.partial(
    pl.kernel,
    mesh=plsc.VectorSubcoreMesh(core_axis_name="c", subcore_axis_name="s"),
    compiler_params=pltpu.CompilerParams(use_tc_tiling_on_sc=False),
    out_type=[
        jax.ShapeDtypeStruct((N_CLS * CTX_LEN, D), jnp.float32),
        jax.ShapeDtypeStruct((N_CLS,), jnp.int32),
    ],
    scratch_types=[
        pltpu.VMEM((CPW, L_PAD), jnp.int32),
        pltpu.VMEM((L_SUF, CPW), jnp.int32),
        pltpu.VMEM((L_PAD, D), jnp.float32),
        pltpu.VMEM((L_PAD, D), jnp.float32),
        pltpu.VMEM((L_PAD, D), jnp.float32),
        pltpu.VMEM((CPW,), jnp.int32),
        pltpu.SemaphoreType.DMA,
        pltpu.SemaphoreType.DMA,
        pltpu.SemaphoreType.DMA,
        pltpu.SemaphoreType.DMA,
        pltpu.SemaphoreType.DMA,
        pltpu.SemaphoreType.DMA,
        pltpu.SemaphoreType.DMA,
    ],
)
def _prompt_kernel(prompt_hbm, promptT_hbm, ctx_hbm, table_hbm, out_hbm,
                   eos_hbm, pbuf, pbufT, r0, r1, r2, eos_v,
                   g0, g1, g2, w0, w1, w2, csem):
    _body(prompt_hbm, promptT_hbm, ctx_hbm, table_hbm, out_hbm, eos_hbm,
          pbuf, pbufT, [r0, r1, r2], eos_v, [g0, g1, g2], [w0, w1, w2], csem)


def kernel(prompt, ctx_embedding, token_embedding):
    prompt_pad = jnp.pad(prompt, ((0, 0), (0, L_PAD - L_SUF)))
    ctx2 = ctx_embedding.reshape(N_CLS * N_CTX, D)
    out2, eos = _prompt_kernel(prompt_pad, prompt.T, ctx2, token_embedding)
    return out2.reshape(N_CLS, CTX_LEN, D), eos


# speedup vs baseline: 1.9058x; 1.9058x over previous
import functools

import jax
import jax.numpy as jnp
from jax import lax
from jax.experimental import pallas as pl
from jax.experimental.pallas import tpu as pltpu
from jax.experimental.pallas import tpu_sc as plsc

N_CLS = 1024
L_SUF = 61
L_PAD = 64
N_CTX = 16
CTX_LEN = 77
D = 512
NC, NS = 2, 16
NW = NC * NS
CPW = N_CLS // NW
NBUF = 3
CBUF = 2


def _prep_body(pr_ref, pad_ref, eos_ref):
    x = pr_ref[:]
    xp = jnp.concatenate(
        [x, jnp.zeros((N_CLS, L_PAD - L_SUF), jnp.int32)], axis=1)
    pad_ref[:] = xp
    pos = lax.broadcasted_iota(jnp.int32, (N_CLS, L_PAD), 1)
    xm = jnp.where(pos < L_SUF, xp, -1)
    mx = jnp.max(xm, axis=1, keepdims=True)
    idx = jnp.min(jnp.where(xm == mx, pos, L_PAD), axis=1, keepdims=True)
    eos_ref[:] = idx + N_CTX


_prep = pl.pallas_call(
    _prep_body,
    out_shape=[
        jax.ShapeDtypeStruct((N_CLS, L_PAD), jnp.int32),
        jax.ShapeDtypeStruct((N_CLS, 1), jnp.int32),
    ],
)


def _body(prompt_hbm, ctx_hbm, table_hbm, out_hbm,
          pbuf, rows, ctxv, gsems, wsems, cisems, cosems):
    wid = lax.axis_index("s") * NC + lax.axis_index("c")
    base = wid * CPW
    pltpu.sync_copy(prompt_hbm.at[pl.ds(base, CPW)], pbuf)

    def issue_gather(t, b):
        pltpu.async_copy(table_hbm.at[pbuf.at[t]], rows[b], gsems[b])

    def wait_gather(t, b):
        pltpu.make_async_copy(table_hbm.at[pbuf.at[t]], rows[b],
                              gsems[b]).wait()

    def issue_ctx(t, cb):
        pltpu.async_copy(ctx_hbm.at[base + t], ctxv[cb], cisems[cb])

    def wait_ctx(t, cb):
        pltpu.make_async_copy(ctx_hbm.at[base + t], ctxv[cb],
                              cisems[cb]).wait()

    def issue_writes(t, b, cb):
        c = base + t
        pltpu.async_copy(rows[b].at[pl.ds(0, 1)],
                         out_hbm.at[c, pl.ds(0, 1)], wsems[b])
        pltpu.async_copy(rows[b].at[pl.ds(1, L_SUF - 1)],
                         out_hbm.at[c, pl.ds(1 + N_CTX, L_SUF - 1)], wsems[b])
        pltpu.async_copy(ctxv[cb], out_hbm.at[c, pl.ds(1, N_CTX)], cosems[cb])

    def drain_writes(b):
        pltpu.make_async_copy(rows[b].at[pl.ds(0, L_SUF)],
                              out_hbm.at[0, pl.ds(0, L_SUF)], wsems[b]).wait()

    def drain_ctx_write(cb):
        pltpu.make_async_copy(ctxv[cb], out_hbm.at[0, pl.ds(1, N_CTX)],
                              cosems[cb]).wait()

    for t in range(NBUF):
        issue_gather(t, t)
    for t in range(CBUF):
        issue_ctx(t, t)

    for t in range(CPW):
        b, cb = t % NBUF, t % CBUF
        wait_gather(t, b)
        wait_ctx(t, cb)
        issue_writes(t, b, cb)
        u = t + 1
        if u < CPW:
            if u + NBUF - 1 < CPW:
                bu = (u + NBUF - 1) % NBUF
                if u - 1 >= 0:
                    drain_writes(bu)
                issue_gather(u + NBUF - 1, bu)
            if u + CBUF - 1 < CPW:
                cbu = (u + CBUF - 1) % CBUF
                drain_ctx_write(cbu)
                issue_ctx(u + CBUF - 1, cbu)

    for t in range(CPW - NBUF, CPW):
        drain_writes(t % NBUF)
    for t in range(CPW - CBUF, CPW):
        drain_ctx_write(t % CBUF)


@functools.partial(
    pl.kernel,
    mesh=plsc.VectorSubcoreMesh(core_axis_name="c", subcore_axis_name="s"),
    compiler_params=pltpu.CompilerParams(use_tc_tiling_on_sc=False),
    out_type=jax.ShapeDtypeStruct((N_CLS, CTX_LEN, D), jnp.float32),
    scratch_types=[
        pltpu.VMEM((CPW, L_PAD), jnp.int32),
        pltpu.VMEM((L_PAD, D), jnp.float32),
        pltpu.VMEM((L_PAD, D), jnp.float32),
        pltpu.VMEM((L_PAD, D), jnp.float32),
        pltpu.VMEM((N_CTX, D), jnp.float32),
        pltpu.VMEM((N_CTX, D), jnp.float32),
        pltpu.SemaphoreType.DMA,
        pltpu.SemaphoreType.DMA,
        pltpu.SemaphoreType.DMA,
        pltpu.SemaphoreType.DMA,
        pltpu.SemaphoreType.DMA,
        pltpu.SemaphoreType.DMA,
        pltpu.SemaphoreType.DMA,
        pltpu.SemaphoreType.DMA,
        pltpu.SemaphoreType.DMA,
        pltpu.SemaphoreType.DMA,
    ],
)
def _prompt_kernel(prompt_hbm, ctx_hbm, table_hbm, out_hbm,
                   pbuf, r0, r1, r2, c0, c1,
                   g0, g1, g2, w0, w1, w2, ci0, ci1, co0, co1):
    _body(prompt_hbm, ctx_hbm, table_hbm, out_hbm,
          pbuf, [r0, r1, r2], [c0, c1],
          [g0, g1, g2], [w0, w1, w2], [ci0, ci1], [co0, co1])


def kernel(prompt, ctx_embedding, token_embedding):
    prompt_pad, eos2 = _prep(prompt)
    out = _prompt_kernel(prompt_pad, ctx_embedding, token_embedding)
    return out, eos2.reshape(N_CLS)


# speedup vs baseline: 2.4988x; 1.3112x over previous
import functools

import jax
import jax.numpy as jnp
from jax import lax
from jax.experimental import pallas as pl
from jax.experimental.pallas import tpu as pltpu
from jax.experimental.pallas import tpu_sc as plsc

N_CLS = 1024
L_SUF = 61
L_PAD = 64
N_CTX = 16
CTX_LEN = 77
D = 512
NC, NS = 2, 16
NW = NC * NS
CPW = N_CLS // NW
HALF = L_PAD // 2
UNITS = CPW * 2
NBUF = 6
AHEAD = 3
G = 16


def _prep_body(pr_ref, pad_ref, eos_ref):
    x = pr_ref[:]
    xp = jnp.concatenate(
        [x, jnp.zeros((N_CLS, L_PAD - L_SUF), jnp.int32)], axis=1)
    pad_ref[:] = xp
    pos = lax.broadcasted_iota(jnp.int32, (N_CLS, L_PAD), 1)
    xm = jnp.where(pos < L_SUF, xp, -1)
    mx = jnp.max(xm, axis=1, keepdims=True)
    idx = jnp.min(jnp.where(xm == mx, pos, L_PAD), axis=1, keepdims=True)
    eos_ref[:] = idx + N_CTX


_prep = pl.pallas_call(
    _prep_body,
    out_shape=[
        jax.ShapeDtypeStruct((N_CLS, L_PAD), jnp.int32),
        jax.ShapeDtypeStruct((N_CLS, 1), jnp.int32),
    ],
)


def _gather_sc(prompt_hbm, table_hbm, emb_hbm, pbuf, rows, gsems, wsems):
    wid = lax.axis_index("s") * NC + lax.axis_index("c")
    base = wid * CPW
    pltpu.sync_copy(prompt_hbm.at[pl.ds(base, CPW)], pbuf)

    def idx_ref(u):
        return pbuf.at[u // 2, pl.ds((u % 2) * HALF, HALF)]

    def issue_gather(u, b):
        pltpu.async_copy(table_hbm.at[idx_ref(u)], rows[b], gsems[b])

    def wait_gather(u, b):
        pltpu.make_async_copy(table_hbm.at[idx_ref(u)], rows[b],
                              gsems[b]).wait()

    def issue_write(u, b):
        dst = emb_hbm.at[base + u // 2, pl.ds((u % 2) * HALF, HALF)]
        pltpu.async_copy(rows[b], dst, wsems[b])

    def drain_write(b):
        pltpu.make_async_copy(rows[b], emb_hbm.at[0, pl.ds(0, HALF)],
                              wsems[b]).wait()

    for u in range(AHEAD):
        issue_gather(u, u % NBUF)
    for t in range(UNITS):
        b = t % NBUF
        wait_gather(t, b)
        issue_write(t, b)
        v = t + AHEAD
        if v < UNITS:
            bv = v % NBUF
            if v >= NBUF:
                drain_write(bv)
            issue_gather(v, bv)
    for t in range(UNITS - NBUF, UNITS):
        drain_write(t % NBUF)


@functools.partial(
    pl.kernel,
    mesh=plsc.VectorSubcoreMesh(core_axis_name="c", subcore_axis_name="s"),
    out_type=jax.ShapeDtypeStruct((N_CLS, L_PAD, D), jnp.float32),
    scratch_types=[
        pltpu.VMEM((CPW, L_PAD), jnp.int32),
        pltpu.VMEM((HALF, D), jnp.float32),
        pltpu.VMEM((HALF, D), jnp.float32),
        pltpu.VMEM((HALF, D), jnp.float32),
        pltpu.VMEM((HALF, D), jnp.float32),
        pltpu.VMEM((HALF, D), jnp.float32),
        pltpu.VMEM((HALF, D), jnp.float32),
        pltpu.SemaphoreType.DMA,
        pltpu.SemaphoreType.DMA,
        pltpu.SemaphoreType.DMA,
        pltpu.SemaphoreType.DMA,
        pltpu.SemaphoreType.DMA,
        pltpu.SemaphoreType.DMA,
        pltpu.SemaphoreType.DMA,
        pltpu.SemaphoreType.DMA,
        pltpu.SemaphoreType.DMA,
        pltpu.SemaphoreType.DMA,
        pltpu.SemaphoreType.DMA,
        pltpu.SemaphoreType.DMA,
    ],
)
def _gather_kernel(prompt_hbm, table_hbm, emb_hbm,
                   pbuf, r0, r1, r2, r3, r4, r5,
                   g0, g1, g2, g3, g4, g5, w0, w1, w2, w3, w4, w5):
    _gather_sc(prompt_hbm, table_hbm, emb_hbm, pbuf,
               [r0, r1, r2, r3, r4, r5],
               [g0, g1, g2, g3, g4, g5], [w0, w1, w2, w3, w4, w5])


def _splice_body(emb_ref, ctx_ref, out_ref):
    out_ref[:, 0:1] = emb_ref[:, 0:1]
    out_ref[:, 1:1 + N_CTX] = ctx_ref[:]
    out_ref[:, 1 + N_CTX:CTX_LEN] = emb_ref[:, 1:L_SUF]


_splice = pl.pallas_call(
    _splice_body,
    grid=(N_CLS // G,),
    in_specs=[
        pl.BlockSpec((G, L_PAD, D), lambda i: (i, 0, 0)),
        pl.BlockSpec((G, N_CTX, D), lambda i: (i, 0, 0)),
    ],
    out_specs=pl.BlockSpec((G, CTX_LEN, D), lambda i: (i, 0, 0)),
    out_shape=jax.ShapeDtypeStruct((N_CLS, CTX_LEN, D), jnp.float32),
)


def kernel(prompt, ctx_embedding, token_embedding):
    prompt_pad, eos2 = _prep(prompt)
    emb = _gather_kernel(prompt_pad, token_embedding)
    out = _splice(emb, ctx_embedding)
    return out, eos2.reshape(N_CLS)


# speedup vs baseline: 2.5218x; 1.0092x over previous
import functools

import jax
import jax.numpy as jnp
from jax import lax
from jax.experimental import pallas as pl
from jax.experimental.pallas import tpu as pltpu
from jax.experimental.pallas import tpu_sc as plsc

N_CLS = 1024
L_SUF = 61
L_PAD = 64
N_CTX = 16
CTX_LEN = 77
D = 512
NC, NS = 2, 16
NW = NC * NS
NSPLIT = 4
SCLS = N_CLS // NSPLIT
CPW = SCLS // NW
HALF = L_PAD // 2
UNITS = CPW * 2
NBUF = 6
AHEAD = 3
G = 16


def _prep_body(pr_ref, pad_ref, eos_ref):
    x = pr_ref[:]
    xp = jnp.concatenate(
        [x, jnp.zeros((N_CLS, L_PAD - L_SUF), jnp.int32)], axis=1)
    pad_ref[:] = xp
    pos = lax.broadcasted_iota(jnp.int32, (N_CLS, L_PAD), 1)
    xm = jnp.where(pos < L_SUF, xp, -1)
    mx = jnp.max(xm, axis=1, keepdims=True)
    idx = jnp.min(jnp.where(xm == mx, pos, L_PAD), axis=1, keepdims=True)
    eos_ref[:] = idx + N_CTX


_prep = pl.pallas_call(
    _prep_body,
    out_shape=[
        jax.ShapeDtypeStruct((N_CLS, L_PAD), jnp.int32),
        jax.ShapeDtypeStruct((N_CLS, 1), jnp.int32),
    ],
)


def _gather_sc(off, prompt_hbm, table_hbm, emb_hbm, pbuf, rows, gsems, wsems):
    wid = lax.axis_index("s") * NC + lax.axis_index("c")
    lbase = wid * CPW
    pltpu.sync_copy(prompt_hbm.at[pl.ds(off + lbase, CPW)], pbuf)

    def idx_ref(u):
        return pbuf.at[u // 2, pl.ds((u % 2) * HALF, HALF)]

    def issue_gather(u, b):
        pltpu.async_copy(table_hbm.at[idx_ref(u)], rows[b], gsems[b])

    def wait_gather(u, b):
        pltpu.make_async_copy(table_hbm.at[idx_ref(u)], rows[b],
                              gsems[b]).wait()

    def issue_write(u, b):
        dst = emb_hbm.at[lbase + u // 2, pl.ds((u % 2) * HALF, HALF)]
        pltpu.async_copy(rows[b], dst, wsems[b])

    def drain_write(b):
        pltpu.make_async_copy(rows[b], emb_hbm.at[0, pl.ds(0, HALF)],
                              wsems[b]).wait()

    for u in range(AHEAD):
        issue_gather(u, u % NBUF)
    for t in range(UNITS):
        b = t % NBUF
        wait_gather(t, b)
        issue_write(t, b)
        v = t + AHEAD
        if v < UNITS:
            bv = v % NBUF
            if v >= NBUF:
                drain_write(bv)
            issue_gather(v, bv)
    for t in range(UNITS - NBUF, UNITS):
        drain_write(t % NBUF)


def _make_gather(off):
    @functools.partial(
        pl.kernel,
        mesh=plsc.VectorSubcoreMesh(core_axis_name="c", subcore_axis_name="s"),
        out_type=jax.ShapeDtypeStruct((SCLS, L_PAD, D), jnp.float32),
        scratch_types=[
            pltpu.VMEM((CPW, L_PAD), jnp.int32),
        ] + [pltpu.VMEM((HALF, D), jnp.float32)] * NBUF
          + [pltpu.SemaphoreType.DMA] * (2 * NBUF),
    )
    def _gather_kernel(prompt_hbm, table_hbm, emb_hbm, pbuf, *rest):
        rows = list(rest[:NBUF])
        gsems = list(rest[NBUF:2 * NBUF])
        wsems = list(rest[2 * NBUF:])
        _gather_sc(off, prompt_hbm, table_hbm, emb_hbm,
                   pbuf, rows, gsems, wsems)

    return _gather_kernel


_gathers = [_make_gather(k * SCLS) for k in range(NSPLIT)]


def _splice_first_body(emb_ref, ctx_ref, out_ref):
    out_ref[:, 0:1] = emb_ref[:, 0:1]
    out_ref[:, 1:1 + N_CTX] = ctx_ref[:]
    out_ref[:, 1 + N_CTX:CTX_LEN] = emb_ref[:, 1:L_SUF]


def _splice_chain_body(prev_ref, emb_ref, ctx_ref, out_ref):
    del prev_ref
    out_ref[:, 0:1] = emb_ref[:, 0:1]
    out_ref[:, 1:1 + N_CTX] = ctx_ref[:]
    out_ref[:, 1 + N_CTX:CTX_LEN] = emb_ref[:, 1:L_SUF]


def _make_splice(off, first):
    blk = off // G
    body = _splice_first_body if first else _splice_chain_body
    in_specs = [
        pl.BlockSpec((G, L_PAD, D), lambda i: (i, 0, 0)),
        pl.BlockSpec((G, N_CTX, D), lambda i, b=blk: (i + b, 0, 0)),
    ]
    if not first:
        in_specs = [pl.BlockSpec(memory_space=pl.ANY)] + in_specs
    return pl.pallas_call(
        body,
        grid=(SCLS // G,),
        in_specs=in_specs,
        out_specs=pl.BlockSpec((G, CTX_LEN, D), lambda i, b=blk: (i + b, 0, 0)),
        out_shape=jax.ShapeDtypeStruct((N_CLS, CTX_LEN, D), jnp.float32),
        input_output_aliases={} if first else {0: 0},
    )


_splices = [_make_splice(k * SCLS, k == 0) for k in range(NSPLIT)]


def kernel(prompt, ctx_embedding, token_embedding):
    prompt_pad, eos2 = _prep(prompt)
    embs = [_gathers[k](prompt_pad, token_embedding) for k in range(NSPLIT)]
    out = _splices[0](embs[0], ctx_embedding)
    for k in range(1, NSPLIT):
        out = _splices[k](out, embs[k], ctx_embedding)
    return out, eos2.reshape(N_CLS)


# speedup vs baseline: 8.5906x; 3.4065x over previous
import functools

import jax
import jax.numpy as jnp
from jax import lax
from jax.experimental import pallas as pl
from jax.experimental.pallas import tpu as pltpu
from jax.experimental.pallas import tpu_sc as plsc

N_CLS = 1024
L_SUF = 61
L_PAD = 64
N_CTX = 16
CTX_LEN = 77
D = 512
NC, NS = 2, 16
NW = NC * NS
CPW = N_CLS // NW
UNITS = L_SUF + N_CTX
NBUF = 6
AHEAD = 3


def _prep_body(pr_ref, eos_ref):
    x = pr_ref[:]
    pos = lax.broadcasted_iota(jnp.int32, (N_CLS, L_SUF), 1)
    mx = jnp.max(x, axis=1, keepdims=True)
    idx = jnp.min(jnp.where(x == mx, pos, L_SUF), axis=1, keepdims=True)
    eos_ref[:] = idx + N_CTX


_prep = pl.pallas_call(
    _prep_body,
    out_shape=jax.ShapeDtypeStruct((N_CLS, 1), jnp.int32),
)


def _gather_sc(idx_hbm, table_hbm, ctx_hbm, out_hbm, ibuf, rows, isems, wsems):
    wid = lax.axis_index("s") * NC + lax.axis_index("c")
    c0 = wid * CPW
    pltpu.sync_copy(idx_hbm.at[pl.ds(wid * (L_SUF * CPW), L_SUF * CPW)], ibuf)

    def src_ref(u):
        if u < L_SUF:
            return table_hbm.at[ibuf.at[pl.ds(u * CPW, CPW)]]
        return ctx_hbm.at[u - L_SUF, pl.ds(c0, CPW)]

    def out_row(u):
        if u == 0:
            return 0
        if u < L_SUF:
            return u + N_CTX
        return u - L_SUF + 1

    def issue_in(u, b):
        pltpu.async_copy(src_ref(u), rows[b], isems[b])

    def wait_in(u, b):
        pltpu.make_async_copy(src_ref(u), rows[b], isems[b]).wait()

    def issue_write(u, b):
        dst = out_hbm.at[out_row(u), pl.ds(c0, CPW)]
        pltpu.async_copy(rows[b], dst, wsems[b])

    def drain_write(b):
        pltpu.make_async_copy(rows[b], out_hbm.at[0, pl.ds(0, CPW)],
                              wsems[b]).wait()

    for u in range(AHEAD):
        issue_in(u, u % NBUF)
    for t in range(UNITS):
        b = t % NBUF
        wait_in(t, b)
        issue_write(t, b)
        v = t + AHEAD
        if v < UNITS:
            bv = v % NBUF
            if v >= NBUF:
                drain_write(bv)
            issue_in(v, bv)
    for t in range(UNITS - NBUF, UNITS):
        drain_write(t % NBUF)


@functools.partial(
    pl.kernel,
    mesh=plsc.VectorSubcoreMesh(core_axis_name="c", subcore_axis_name="s"),
    out_type=jax.ShapeDtypeStruct((CTX_LEN, N_CLS, D), jnp.float32),
    scratch_types=[
        pltpu.VMEM((L_SUF * CPW,), jnp.int32),
    ] + [pltpu.VMEM((CPW, D), jnp.float32)] * NBUF
      + [pltpu.SemaphoreType.DMA] * (2 * NBUF),
)
def _gather_kernel(idx_hbm, table_hbm, ctx_hbm, out_hbm, ibuf, *rest):
    _gather_sc(idx_hbm, table_hbm, ctx_hbm, out_hbm, ibuf,
               list(rest[:NBUF]), list(rest[NBUF:2 * NBUF]),
               list(rest[2 * NBUF:]))


def kernel(prompt, ctx_embedding, token_embedding):
    eos2 = _prep(prompt)
    pad = jnp.pad(prompt, ((0, 0), (0, L_PAD - L_SUF)))
    idx_flat = (pad.reshape(NW, CPW, L_PAD)[:, :, :L_SUF]
                .transpose(0, 2, 1).reshape(-1))
    ctx_t = jnp.transpose(ctx_embedding, (1, 0, 2))
    outp = _gather_kernel(idx_flat, token_embedding, ctx_t)
    return jnp.transpose(outp, (1, 0, 2)), eos2.reshape(N_CLS)
